# baseline (speedup 1.0000x reference)
import jax
import jax.numpy as jnp
from jax import lax
from jax.experimental import pallas as pl
from jax.experimental.pallas import tpu as pltpu
from jax.experimental.pallas import tpu_sc as plsc

_B = 512
_C = 512
_HW = 196
_G = 1024
_TB = 64
_EPS = 1e-8
_AVG = 0.5
_L = 16
_CH = 128
_NCH = _G // _CH
_NP = _NCH // 2


def _tc_body(maps_ref, w1_ref, w2_ref, xsun_ref, xgl_ref, votet_ref,
             dense_ref):
    x = maps_ref[...]
    s = jnp.sum(x, axis=0) * (1.0 / _HW)
    xsun_ref[...] = s
    xlog = jnp.log(jnp.tanh(jnp.maximum(s, 0.0) + _EPS))
    gl = jax.lax.dot_general(
        xlog, w1_ref[...], (((1,), (1,)), ((), ())),
        preferred_element_type=jnp.float32)
    xgl_ref[...] = gl
    vote = (jnp.exp(gl) - _EPS) * w2_ref[...]
    dense_ref[...] = jnp.sum(vote, axis=1, keepdims=True) + _AVG
    votet_ref[0, :, :] = vote.T


def _sc_body(votet_ref, outt_ref, b0, b1, b2, b3, obuf, s0, s1, s2, s3):
    wid = lax.axis_index("s") * 2 + lax.axis_index("c")
    nblocks = votet_ref.shape[0]
    nb = wid // (_TB // _L)
    off = (wid % (_TB // _L)) * _L
    bufs = (b0, b1, b2, b3)
    sems = (s0, s1, s2, s3)

    @pl.when(nb < nblocks)
    def _run():
        _sc_scan(votet_ref, outt_ref, wid, nb, off, bufs, sems, obuf)


def _sc_scan(votet_ref, outt_ref, wid, nb, off, bufs, sems, obuf):

    def chunk_copy(c, slot):
        return pltpu.async_copy(
            votet_ref.at[nb, pl.ds(c * _CH, _CH)], bufs[slot], sems[slot])

    def make_step(ref_a, ref_b):
        def ins(ref, i, ks, vs):
            x = ref[i, pl.ds(off, _L)]
            ck, cv = jnp.abs(x), x
            nks, nvs = [], []
            for j in range(8):
                gt = ck > ks[j]
                nks.append(jnp.where(gt, ck, ks[j]))
                nvs.append(jnp.where(gt, cv, vs[j]))
                ck, cv = jnp.where(gt, ks[j], ck), jnp.where(gt, vs[j], cv)
            return nks + nvs

        def step(i, carry):
            return tuple(ins(ref_a, i, carry[:8], carry[8:16])) + tuple(
                ins(ref_b, i, carry[16:24], carry[24:32]))
        return step

    neg = jnp.full((_L,), -1.0, jnp.float32)
    z = jnp.zeros((_L,), jnp.float32)
    carry = (neg,) * 8 + (z,) * 8 + (neg,) * 8 + (z,) * 8
    h = {0: chunk_copy(0, 0), 1: chunk_copy(_NP, 1)}
    for p in range(_NP):
        if p + 1 < _NP:
            nslot = 2 * ((p + 1) % 2)
            h[2 * (p + 1)] = chunk_copy(p + 1, nslot)
            h[2 * (p + 1) + 1] = chunk_copy(_NP + p + 1, nslot + 1)
        h[2 * p].wait()
        h[2 * p + 1].wait()
        slot = 2 * (p % 2)
        carry = lax.fori_loop(
            0, _CH, make_step(bufs[slot], bufs[slot + 1]), carry)
    ka, va = list(carry[:8]), list(carry[8:16])
    for j in range(8):
        ck, cv = carry[16 + j], carry[24 + j]
        for t in range(8):
            gt = ck > ka[t]
            nk = jnp.where(gt, ck, ka[t])
            nv = jnp.where(gt, cv, va[t])
            ck, cv = jnp.where(gt, ka[t], ck), jnp.where(gt, va[t], cv)
            ka[t], va[t] = nk, nv
    acc = z
    for j in range(8):
        acc = acc + va[j]
        obuf[j] = acc + _AVG
    pltpu.sync_copy(obuf, outt_ref.at[wid])


def _sc_topk(votet):
    nrows = votet.shape[0] * _TB
    mesh = plsc.VectorSubcoreMesh(
        core_axis_name="c", subcore_axis_name="s",
        num_cores=2, num_subcores=16)
    return pl.kernel(
        _sc_body,
        out_type=jax.ShapeDtypeStruct((nrows // _L, 8, _L), jnp.float32),
        mesh=mesh,
        scratch_types=[
            pltpu.VMEM((_CH, _TB), jnp.float32),
            pltpu.VMEM((_CH, _TB), jnp.float32),
            pltpu.VMEM((_CH, _TB), jnp.float32),
            pltpu.VMEM((_CH, _TB), jnp.float32),
            pltpu.VMEM((8, _L), jnp.float32),
            pltpu.SemaphoreType.DMA,
            pltpu.SemaphoreType.DMA,
            pltpu.SemaphoreType.DMA,
            pltpu.SemaphoreType.DMA,
        ],
    )(votet)


def _tc_half(maps_t, W1, W2, half):
    hb = _B // 2
    base = half * (hb // _TB)
    return pl.pallas_call(
        _tc_body,
        grid=(hb // _TB,),
        in_specs=[
            pl.BlockSpec((_HW, _TB, _C), lambda i: (0, base + i, 0)),
            pl.BlockSpec((_G, _C), lambda i: (0, 0)),
            pl.BlockSpec((1, _G), lambda i: (0, 0)),
        ],
        out_specs=[
            pl.BlockSpec((_TB, _C), lambda i: (i, 0)),
            pl.BlockSpec((_TB, _G), lambda i: (i, 0)),
            pl.BlockSpec((1, _G, _TB), lambda i: (i, 0, 0)),
            pl.BlockSpec((_TB, 1), lambda i: (i, 0)),
        ],
        out_shape=[
            jax.ShapeDtypeStruct((hb, _C), jnp.float32),
            jax.ShapeDtypeStruct((hb, _G), jnp.float32),
            jax.ShapeDtypeStruct((hb // _TB, _G, _TB), jnp.float32),
            jax.ShapeDtypeStruct((hb, 1), jnp.float32),
        ],
    )(maps_t, W1, W2)


def kernel(maps, W1, W2):
    maps_t = maps.transpose(2, 3, 0, 1).reshape(_HW, _B, _C)
    xsun0, xgl0, votet0, dense0 = _tc_half(maps_t, W1, W2, 0)
    topk30 = _sc_topk(votet0)
    xsun1, xgl1, votet1, dense1 = _tc_half(maps_t, W1, W2, 1)
    topk31 = _sc_topk(votet1)
    xsun = jnp.concatenate([xsun0, xsun1], axis=0)
    xgl = jnp.concatenate([xgl0, xgl1], axis=0)
    dense = jnp.concatenate([dense0, dense1], axis=0)
    topk3 = jnp.concatenate([topk30, topk31], axis=0)
    topk = topk3.transpose(0, 2, 1).reshape(_B, 8)
    xson = jnp.concatenate([topk, dense], axis=1)
    return (xsun, xgl, xson)


# speedup vs baseline: 1.0711x; 1.0711x over previous
import jax
import jax.numpy as jnp
from jax import lax
from jax.experimental import pallas as pl
from jax.experimental.pallas import tpu as pltpu
from jax.experimental.pallas import tpu_sc as plsc

_B = 512
_C = 512
_HW = 196
_G = 1024
_TB = 64
_EPS = 1e-8
_AVG = 0.5
_L = 16
_CH = 128
_NCH = _G // _CH
_NP = _NCH // 2


def _tc_body(maps_ref, w1_ref, w2_ref, xsun_ref, xgl_ref, votet_ref,
             dense_ref):
    x = maps_ref[...]
    s = jnp.sum(x, axis=0) * (1.0 / _HW)
    xsun_ref[...] = s
    xlog = jnp.log(jnp.tanh(jnp.maximum(s, 0.0) + _EPS))
    gl = jax.lax.dot_general(
        xlog, w1_ref[...], (((1,), (1,)), ((), ())),
        preferred_element_type=jnp.float32)
    xgl_ref[...] = gl
    vote = (jnp.exp(gl) - _EPS) * w2_ref[...]
    dense_ref[...] = jnp.sum(vote, axis=1, keepdims=True) + _AVG
    votet_ref[0, :, :] = vote.T


def _sc_body(votet_ref, outt_ref, b0, b1, b2, b3, obuf, s0, s1, s2, s3):
    wid = lax.axis_index("s") * 2 + lax.axis_index("c")
    nb = wid // (_TB // _L)
    off = (wid % (_TB // _L)) * _L
    bufs = (b0, b1, b2, b3)
    sems = (s0, s1, s2, s3)

    def chunk_copy(c, slot):
        return pltpu.async_copy(
            votet_ref.at[nb, pl.ds(c * _CH, _CH)], bufs[slot], sems[slot])

    def make_step(ref_a, ref_b):
        def ins(ref, i, ks, vs):
            x = ref[i, pl.ds(off, _L)]
            ck, cv = jnp.abs(x), x
            nks, nvs = [], []
            for j in range(8):
                gt = ck > ks[j]
                nks.append(jnp.where(gt, ck, ks[j]))
                nvs.append(jnp.where(gt, cv, vs[j]))
                ck, cv = jnp.where(gt, ks[j], ck), jnp.where(gt, vs[j], cv)
            return nks + nvs

        def step(i, carry):
            return tuple(ins(ref_a, i, carry[:8], carry[8:16])) + tuple(
                ins(ref_b, i, carry[16:24], carry[24:32]))
        return step

    neg = jnp.full((_L,), -1.0, jnp.float32)
    z = jnp.zeros((_L,), jnp.float32)
    carry = (neg,) * 8 + (z,) * 8 + (neg,) * 8 + (z,) * 8
    h = {0: chunk_copy(0, 0), 1: chunk_copy(_NP, 1)}
    for p in range(_NP):
        if p + 1 < _NP:
            nslot = 2 * ((p + 1) % 2)
            h[2 * (p + 1)] = chunk_copy(p + 1, nslot)
            h[2 * (p + 1) + 1] = chunk_copy(_NP + p + 1, nslot + 1)
        h[2 * p].wait()
        h[2 * p + 1].wait()
        slot = 2 * (p % 2)
        carry = lax.fori_loop(
            0, _CH, make_step(bufs[slot], bufs[slot + 1]), carry)
    ka, va = list(carry[:8]), list(carry[8:16])
    for j in range(8):
        ck, cv = carry[16 + j], carry[24 + j]
        for t in range(8):
            gt = ck > ka[t]
            nk = jnp.where(gt, ck, ka[t])
            nv = jnp.where(gt, cv, va[t])
            ck, cv = jnp.where(gt, ka[t], ck), jnp.where(gt, va[t], cv)
            ka[t], va[t] = nk, nv
    acc = z
    for j in range(8):
        acc = acc + va[j]
        obuf[j] = acc + _AVG
    pltpu.sync_copy(obuf, outt_ref.at[wid])


def _sc_topk(votet):
    mesh = plsc.VectorSubcoreMesh(
        core_axis_name="c", subcore_axis_name="s",
        num_cores=2, num_subcores=16)
    return pl.kernel(
        _sc_body,
        out_type=jax.ShapeDtypeStruct((_B // _L, 8, _L), jnp.float32),
        mesh=mesh,
        scratch_types=[
            pltpu.VMEM((_CH, _TB), jnp.float32),
            pltpu.VMEM((_CH, _TB), jnp.float32),
            pltpu.VMEM((_CH, _TB), jnp.float32),
            pltpu.VMEM((_CH, _TB), jnp.float32),
            pltpu.VMEM((8, _L), jnp.float32),
            pltpu.SemaphoreType.DMA,
            pltpu.SemaphoreType.DMA,
            pltpu.SemaphoreType.DMA,
            pltpu.SemaphoreType.DMA,
        ],
    )(votet)


def kernel(maps, W1, W2):
    maps_t = maps.transpose(2, 3, 0, 1).reshape(_HW, _B, _C)
    xsun, xgl, votet, dense = pl.pallas_call(
        _tc_body,
        grid=(_B // _TB,),
        in_specs=[
            pl.BlockSpec((_HW, _TB, _C), lambda i: (0, i, 0)),
            pl.BlockSpec((_G, _C), lambda i: (0, 0)),
            pl.BlockSpec((1, _G), lambda i: (0, 0)),
        ],
        out_specs=[
            pl.BlockSpec((_TB, _C), lambda i: (i, 0)),
            pl.BlockSpec((_TB, _G), lambda i: (i, 0)),
            pl.BlockSpec((1, _G, _TB), lambda i: (i, 0, 0)),
            pl.BlockSpec((_TB, 1), lambda i: (i, 0)),
        ],
        out_shape=[
            jax.ShapeDtypeStruct((_B, _C), jnp.float32),
            jax.ShapeDtypeStruct((_B, _G), jnp.float32),
            jax.ShapeDtypeStruct((_B // _TB, _G, _TB), jnp.float32),
            jax.ShapeDtypeStruct((_B, 1), jnp.float32),
        ],
    )(maps_t, W1, W2)
    topk3 = _sc_topk(votet)
    topk = topk3.transpose(0, 2, 1).reshape(_B, 8)
    xson = jnp.concatenate([topk, dense], axis=1)
    return (xsun, xgl, xson)
